# Initial kernel scaffold; baseline (speedup 1.0000x reference)
import functools

import jax
import jax.numpy as jnp
from jax import lax
from jax.experimental import pallas as pl
from jax.experimental.pallas import tpu as pltpu
from jax.experimental.pallas import tpu_sc as plsc

NUM_USERS = 1000000
NUM_SKILLS = 100000
EMBED_DIM = 32
BATCH = 16384
HIST = 50

NC = 2
NS = 16
NW = NC * NS

U_PER_W = BATCH // NW
SK_TOTAL = BATCH * HIST
S_PER_W = SK_TOTAL // NW
S_CHUNK = 512
S_STEPS = S_PER_W // S_CHUNK


def _gather_body(u_idx_hbm, s_idx_hbm, user_table, skill_table,
                 u_out, s_out, u_idx_v, u_rows, s_idx_v, s_rows, sem):
    wid = lax.axis_index("s") * NC + lax.axis_index("c")

    u_base = wid * U_PER_W
    pltpu.sync_copy(u_idx_hbm.at[pl.ds(u_base, U_PER_W)], u_idx_v)
    pltpu.async_copy(user_table.at[u_idx_v], u_rows, sem).wait()
    pltpu.sync_copy(u_rows, u_out.at[pl.ds(u_base, U_PER_W)])

    s_base = wid * S_PER_W

    def step(c, carry):
        off = s_base + c * S_CHUNK
        pltpu.sync_copy(s_idx_hbm.at[pl.ds(off, S_CHUNK)], s_idx_v)
        pltpu.async_copy(skill_table.at[s_idx_v], s_rows, sem).wait()
        pltpu.sync_copy(s_rows, s_out.at[pl.ds(off, S_CHUNK)])
        return carry

    lax.fori_loop(0, S_STEPS, step, 0)


@jax.jit
def _run(user_indices, query_skills_flat, user_table, skill_table):
    mesh = plsc.VectorSubcoreMesh(core_axis_name="c", subcore_axis_name="s",
                                  num_cores=NC, num_subcores=NS)
    return pl.kernel(
        _gather_body,
        out_type=(
            jax.ShapeDtypeStruct((BATCH, EMBED_DIM), jnp.float32),
            jax.ShapeDtypeStruct((SK_TOTAL, EMBED_DIM), jnp.float32),
        ),
        mesh=mesh,
        scratch_types=[
            pltpu.VMEM((U_PER_W,), jnp.int32),
            pltpu.VMEM((U_PER_W, EMBED_DIM), jnp.float32),
            pltpu.VMEM((S_CHUNK,), jnp.int32),
            pltpu.VMEM((S_CHUNK, EMBED_DIM), jnp.float32),
            pltpu.SemaphoreType.DMA,
        ],
    )(user_indices, query_skills_flat, user_table, skill_table)


def kernel(user_indices, query_skills, user_table, skill_table):
    u_idx = user_indices.astype(jnp.int32)
    s_idx = query_skills.astype(jnp.int32).reshape(-1)
    u_emb, s_emb = _run(u_idx, s_idx, user_table, skill_table)
    return (u_emb, s_emb.reshape(BATCH, HIST, EMBED_DIM))


# speedup vs baseline: 2.1303x; 2.1303x over previous
import functools

import jax
import jax.numpy as jnp
from jax import lax
from jax.experimental import pallas as pl
from jax.experimental.pallas import tpu as pltpu
from jax.experimental.pallas import tpu_sc as plsc

NUM_USERS = 1000000
NUM_SKILLS = 100000
EMBED_DIM = 32
BATCH = 16384
HIST = 50

NC = 2
NS = 16
NW = NC * NS

U_PER_W = BATCH // NW
SK_TOTAL = BATCH * HIST
S_PER_W = SK_TOTAL // NW
S_CHUNK = 512
S_STEPS = S_PER_W // S_CHUNK


def _gather_body(u_idx_hbm, s_idx_hbm, user_table, skill_table,
                 u_out, s_out, u_idx_v, u_rows, s_idx_v, s_rows, sem):
    wid = lax.axis_index("s") * NC + lax.axis_index("c")

    u_base = wid * U_PER_W
    pltpu.sync_copy(u_idx_hbm.at[pl.ds(u_base, U_PER_W)], u_idx_v)
    pltpu.async_copy(user_table.at[u_idx_v], u_rows, sem).wait()
    pltpu.sync_copy(u_rows, u_out.at[pl.ds(u_base, U_PER_W)])

    s_base = wid * S_PER_W

    def step(c, carry):
        off = s_base + c * S_CHUNK
        pltpu.sync_copy(s_idx_hbm.at[pl.ds(off, S_CHUNK)], s_idx_v)
        pltpu.async_copy(skill_table.at[s_idx_v], s_rows, sem).wait()
        pltpu.sync_copy(s_rows, s_out.at[pl.ds(off, S_CHUNK)])
        return carry

    lax.fori_loop(0, S_STEPS, step, 0)


@jax.jit
def _run(user_indices, query_skills_flat, user_table, skill_table):
    mesh = plsc.VectorSubcoreMesh(core_axis_name="c", subcore_axis_name="s",
                                  num_cores=NC, num_subcores=NS)
    return pl.kernel(
        _gather_body,
        out_type=(
            jax.ShapeDtypeStruct((BATCH, EMBED_DIM), jnp.float32),
            jax.ShapeDtypeStruct((SK_TOTAL, EMBED_DIM), jnp.float32),
        ),
        mesh=mesh,
        scratch_types=[
            pltpu.VMEM((U_PER_W,), jnp.int32),
            pltpu.VMEM((U_PER_W, EMBED_DIM), jnp.float32),
            pltpu.VMEM((S_CHUNK,), jnp.int32),
            pltpu.VMEM((S_CHUNK, EMBED_DIM), jnp.float32),
            pltpu.SemaphoreType.DMA,
        ],
        compiler_params=pltpu.CompilerParams(use_tc_tiling_on_sc=False),
    )(user_indices, query_skills_flat, user_table, skill_table)


def kernel(user_indices, query_skills, user_table, skill_table):
    u_idx = user_indices.astype(jnp.int32)
    s_idx = query_skills.astype(jnp.int32).reshape(-1)
    u_emb, s_emb = _run(u_idx, s_idx, user_table, skill_table)
    return (u_emb, s_emb.reshape(BATCH, HIST, EMBED_DIM))


# speedup vs baseline: 2.2068x; 1.0359x over previous
import jax
import jax.numpy as jnp
from jax import lax
from jax.experimental import pallas as pl
from jax.experimental.pallas import tpu as pltpu
from jax.experimental.pallas import tpu_sc as plsc

NUM_USERS = 1000000
NUM_SKILLS = 100000
EMBED_DIM = 32
BATCH = 16384
HIST = 50

NC = 2
NS = 16
NW = NC * NS

U_PER_W = BATCH // NW
SK_TOTAL = BATCH * HIST
S_PER_W = SK_TOTAL // NW
S_CHUNK = 512
S_STEPS = S_PER_W // S_CHUNK
NB = 5


def _gather_body(u_idx_hbm, s_idx_hbm, user_table, skill_table,
                 u_out, s_out, s_idx_all, rows, u_idx_v, u_rows,
                 gsems, wsems, usem, uwsem):
    wid = lax.axis_index("s") * NC + lax.axis_index("c")
    u_base = wid * U_PER_W
    s_base = wid * S_PER_W

    def s_gather(c, b):
        return pltpu.make_async_copy(
            skill_table.at[s_idx_all.at[pl.ds(c * S_CHUNK, S_CHUNK)]],
            rows[b], gsems[b])

    def s_write(c, b):
        return pltpu.make_async_copy(
            rows[b], s_out.at[pl.ds(s_base + c * S_CHUNK, S_CHUNK)], wsems[b])

    pltpu.sync_copy(u_idx_hbm.at[pl.ds(u_base, U_PER_W)], u_idx_v)
    pltpu.sync_copy(s_idx_hbm.at[pl.ds(s_base, S_PER_W)], s_idx_all)

    pltpu.make_async_copy(user_table.at[u_idx_v], u_rows, usem).start()
    for b in range(NB):
        s_gather(b, b).start()
    pltpu.make_async_copy(user_table.at[u_idx_v], u_rows, usem).wait()
    u_wr = pltpu.make_async_copy(
        u_rows, u_out.at[pl.ds(u_base, U_PER_W)], uwsem)
    u_wr.start()

    def outer(g, carry):
        for b in range(NB):
            c = g * NB + b
            s_gather(c, b).wait()
            s_write(c, b).start()
            s_write(c, b).wait()
            s_gather(c + NB, b).start()
        return carry

    lax.fori_loop(0, (S_STEPS - NB) // NB, outer, 0)

    for b in range(NB):
        c = S_STEPS - NB + b
        s_gather(c, b).wait()
        s_write(c, b).start()
    for b in range(NB):
        s_write(S_STEPS - NB + b, b).wait()
    u_wr.wait()


@jax.jit
def _run(user_indices, query_skills_flat, user_table, skill_table):
    mesh = plsc.VectorSubcoreMesh(core_axis_name="c", subcore_axis_name="s",
                                  num_cores=NC, num_subcores=NS)
    return pl.kernel(
        lambda ui, si, ut, st, uo, so, sidx, r0, r1, r2, r3, r4, uiv, ur,
               g0, g1, g2, g3, g4, w0, w1, w2, w3, w4, us, uw: _gather_body(
                   ui, si, ut, st, uo, so, sidx, [r0, r1, r2, r3, r4],
                   uiv, ur, [g0, g1, g2, g3, g4], [w0, w1, w2, w3, w4],
                   us, uw),
        out_type=(
            jax.ShapeDtypeStruct((BATCH, EMBED_DIM), jnp.float32),
            jax.ShapeDtypeStruct((SK_TOTAL, EMBED_DIM), jnp.float32),
        ),
        mesh=mesh,
        scratch_types=(
            [pltpu.VMEM((S_PER_W,), jnp.int32)]
            + [pltpu.VMEM((S_CHUNK, EMBED_DIM), jnp.float32)
               for _ in range(NB)]
            + [pltpu.VMEM((U_PER_W,), jnp.int32),
               pltpu.VMEM((U_PER_W, EMBED_DIM), jnp.float32)]
            + [pltpu.SemaphoreType.DMA for _ in range(2 * NB + 2)]
        ),
        compiler_params=pltpu.CompilerParams(use_tc_tiling_on_sc=False),
    )(user_indices, query_skills_flat, user_table, skill_table)


def kernel(user_indices, query_skills, user_table, skill_table):
    u_idx = user_indices.astype(jnp.int32)
    s_idx = query_skills.astype(jnp.int32).reshape(-1)
    u_emb, s_emb = _run(u_idx, s_idx, user_table, skill_table)
    return (u_emb, s_emb.reshape(BATCH, HIST, EMBED_DIM))


# speedup vs baseline: 3.5447x; 1.6063x over previous
import jax
import jax.numpy as jnp
from jax import lax
from jax.experimental import pallas as pl
from jax.experimental.pallas import tpu as pltpu
from jax.experimental.pallas import tpu_sc as plsc

NUM_USERS = 1000000
NUM_SKILLS = 100000
EMBED_DIM = 32
BATCH = 16384
HIST = 50

NC = 2
NS = 16
NW = NC * NS

U_PER_W = BATCH // NW
SK_TOTAL = BATCH * HIST
S_PER_W = SK_TOTAL // NW
USERS_PER_CHUNK = 8
S_CHUNK = USERS_PER_CHUNK * HIST
S_STEPS = S_PER_W // S_CHUNK
NB = 4


def _gather_body(u_idx_hbm, s_idx_hbm, user_table, skill_table,
                 u_out, s_out3, s_idx_all, rows, u_idx_v, u_rows,
                 gsems, wsems, usem, uwsem):
    wid = lax.axis_index("s") * NC + lax.axis_index("c")
    u_base = wid * U_PER_W
    s_base = wid * S_PER_W

    def s_gather(c, b):
        return pltpu.make_async_copy(
            skill_table.at[s_idx_all.at[pl.ds(c * S_CHUNK, S_CHUNK)]],
            rows[b], gsems[b])

    def s_write_start(c, b):
        user0 = u_base + c * USERS_PER_CHUNK
        for u in range(USERS_PER_CHUNK):
            pltpu.make_async_copy(
                rows[b].at[pl.ds(u * HIST, HIST)],
                s_out3.at[user0 + u], wsems[b]).start()

    def s_write_wait(b):
        for u in range(USERS_PER_CHUNK):
            pltpu.make_async_copy(
                rows[b].at[pl.ds(u * HIST, HIST)],
                s_out3.at[0], wsems[b]).wait()

    pltpu.sync_copy(u_idx_hbm.at[pl.ds(u_base, U_PER_W)], u_idx_v)
    pltpu.sync_copy(s_idx_hbm.at[pl.ds(s_base, S_PER_W)], s_idx_all)

    pltpu.make_async_copy(user_table.at[u_idx_v], u_rows, usem).start()
    for b in range(NB):
        s_gather(b, b).start()
    pltpu.make_async_copy(user_table.at[u_idx_v], u_rows, usem).wait()
    u_wr = pltpu.make_async_copy(
        u_rows, u_out.at[pl.ds(u_base, U_PER_W)], uwsem)
    u_wr.start()

    def outer(g, carry):
        for b in range(NB):
            c = g * NB + b
            s_gather(c, b).wait()
            s_write_start(c, b)
            s_write_wait(b)
            s_gather(c + NB, b).start()
        return carry

    lax.fori_loop(0, (S_STEPS - NB) // NB, outer, 0)

    for b in range(NB):
        c = S_STEPS - NB + b
        s_gather(c, b).wait()
        s_write_start(c, b)
    for b in range(NB):
        s_write_wait(b)
    u_wr.wait()


@jax.jit
def _run(user_indices, query_skills_flat, user_table, skill_table):
    mesh = plsc.VectorSubcoreMesh(core_axis_name="c", subcore_axis_name="s",
                                  num_cores=NC, num_subcores=NS)
    return pl.kernel(
        lambda ui, si, ut, st, uo, so, sidx, r0, r1, r2, r3, uiv, ur,
               g0, g1, g2, g3, w0, w1, w2, w3, us, uw: _gather_body(
                   ui, si, ut, st, uo, so, sidx, [r0, r1, r2, r3],
                   uiv, ur, [g0, g1, g2, g3], [w0, w1, w2, w3],
                   us, uw),
        out_type=(
            jax.ShapeDtypeStruct((BATCH, EMBED_DIM), jnp.float32),
            jax.ShapeDtypeStruct((BATCH, HIST, EMBED_DIM), jnp.float32),
        ),
        mesh=mesh,
        scratch_types=(
            [pltpu.VMEM((S_PER_W,), jnp.int32)]
            + [pltpu.VMEM((S_CHUNK, EMBED_DIM), jnp.float32)
               for _ in range(NB)]
            + [pltpu.VMEM((U_PER_W,), jnp.int32),
               pltpu.VMEM((U_PER_W, EMBED_DIM), jnp.float32)]
            + [pltpu.SemaphoreType.DMA for _ in range(2 * NB + 2)]
        ),
        compiler_params=pltpu.CompilerParams(use_tc_tiling_on_sc=False),
    )(user_indices, query_skills_flat, user_table, skill_table)


def kernel(user_indices, query_skills, user_table, skill_table):
    u_idx = user_indices.astype(jnp.int32)
    s_idx = query_skills.astype(jnp.int32).reshape(-1)
    return _run(u_idx, s_idx, user_table, skill_table)


# speedup vs baseline: 3.6343x; 1.0253x over previous
import jax
import jax.numpy as jnp
from jax import lax
from jax.experimental import pallas as pl
from jax.experimental.pallas import tpu as pltpu
from jax.experimental.pallas import tpu_sc as plsc

NUM_USERS = 1000000
NUM_SKILLS = 100000
EMBED_DIM = 32
BATCH = 16384
HIST = 50

NC = 2
NS = 16
NW = NC * NS

U_PER_W = BATCH // NW
SK_TOTAL = BATCH * HIST
S_PER_W = SK_TOTAL // NW
USERS_PER_CHUNK = 8
S_CHUNK = USERS_PER_CHUNK * HIST
S_STEPS = S_PER_W // S_CHUNK
NB = 4


def _mesh():
    return plsc.VectorSubcoreMesh(core_axis_name="c", subcore_axis_name="s",
                                  num_cores=NC, num_subcores=NS)


def _user_body(u_idx_hbm, user_table, u_out, u_idx_v, u_rows, sem):
    wid = lax.axis_index("s") * NC + lax.axis_index("c")
    u_base = wid * U_PER_W
    pltpu.sync_copy(u_idx_hbm.at[pl.ds(u_base, U_PER_W)], u_idx_v)
    pltpu.async_copy(user_table.at[u_idx_v], u_rows, sem).wait()
    pltpu.sync_copy(u_rows, u_out.at[pl.ds(u_base, U_PER_W)])


def _skill_body(s_idx_hbm, skill_table, s_out3, s_idx_all, rows,
                gsems, wsems):
    wid = lax.axis_index("s") * NC + lax.axis_index("c")
    u_base = wid * U_PER_W
    s_base = wid * S_PER_W

    def s_gather(c, b):
        return pltpu.make_async_copy(
            skill_table.at[s_idx_all.at[pl.ds(c * S_CHUNK, S_CHUNK)]],
            rows[b], gsems[b])

    def s_write_start(c, b):
        user0 = u_base + c * USERS_PER_CHUNK
        for u in range(USERS_PER_CHUNK):
            pltpu.make_async_copy(
                rows[b].at[pl.ds(u * HIST, HIST)],
                s_out3.at[user0 + u], wsems[b]).start()

    def s_write_wait(b):
        for u in range(USERS_PER_CHUNK):
            pltpu.make_async_copy(
                rows[b].at[pl.ds(u * HIST, HIST)],
                s_out3.at[0], wsems[b]).wait()

    pltpu.sync_copy(s_idx_hbm.at[pl.ds(s_base, S_PER_W)], s_idx_all)
    for b in range(NB):
        s_gather(b, b).start()

    def outer(g, carry):
        for b in range(NB):
            c = g * NB + b
            s_gather(c, b).wait()
            s_write_start(c, b)
            s_write_wait(b)
            s_gather(c + NB, b).start()
        return carry

    lax.fori_loop(0, (S_STEPS - NB) // NB, outer, 0)

    for b in range(NB):
        c = S_STEPS - NB + b
        s_gather(c, b).wait()
        s_write_start(c, b)
    for b in range(NB):
        s_write_wait(b)


@jax.jit
def _run(user_indices, query_skills_flat, user_table, skill_table):
    s_emb = pl.kernel(
        lambda si, st, so, sidx, r0, r1, r2, r3, g0, g1, g2, g3,
               w0, w1, w2, w3: _skill_body(
                   si, st, so, sidx, [r0, r1, r2, r3],
                   [g0, g1, g2, g3], [w0, w1, w2, w3]),
        out_type=jax.ShapeDtypeStruct((BATCH, HIST, EMBED_DIM), jnp.float32),
        mesh=_mesh(),
        scratch_types=(
            [pltpu.VMEM((S_PER_W,), jnp.int32)]
            + [pltpu.VMEM((S_CHUNK, EMBED_DIM), jnp.float32)
               for _ in range(NB)]
            + [pltpu.SemaphoreType.DMA for _ in range(2 * NB)]
        ),
        compiler_params=pltpu.CompilerParams(use_tc_tiling_on_sc=False),
    )(query_skills_flat, skill_table)

    u_emb = pl.kernel(
        _user_body,
        out_type=jax.ShapeDtypeStruct((BATCH, EMBED_DIM), jnp.float32),
        mesh=_mesh(),
        scratch_types=[
            pltpu.VMEM((U_PER_W,), jnp.int32),
            pltpu.VMEM((U_PER_W, EMBED_DIM), jnp.float32),
            pltpu.SemaphoreType.DMA,
        ],
        compiler_params=pltpu.CompilerParams(use_tc_tiling_on_sc=False),
    )(user_indices, user_table)

    return (u_emb, s_emb)


def kernel(user_indices, query_skills, user_table, skill_table):
    u_idx = user_indices.astype(jnp.int32)
    s_idx = query_skills.astype(jnp.int32).reshape(-1)
    return _run(u_idx, s_idx, user_table, skill_table)


# speedup vs baseline: 3.6363x; 1.0006x over previous
import jax
import jax.numpy as jnp
from jax import lax
from jax.experimental import pallas as pl
from jax.experimental.layout import Format, Layout
from jax.experimental.pallas import tpu as pltpu
from jax.experimental.pallas import tpu_sc as plsc

NUM_USERS = 1000000
NUM_SKILLS = 100000
EMBED_DIM = 32
BATCH = 16384
HIST = 50

NC = 2
NS = 16
NW = NC * NS

U_PER_W = BATCH // NW
SK_TOTAL = BATCH * HIST
S_PER_W = SK_TOTAL // NW
USERS_PER_CHUNK = 8
S_CHUNK = USERS_PER_CHUNK * HIST
S_STEPS = S_PER_W // S_CHUNK
NB = 4


def _mesh():
    return plsc.VectorSubcoreMesh(core_axis_name="c", subcore_axis_name="s",
                                  num_cores=NC, num_subcores=NS)


def _user_body(u_idx_hbm, user_table, u_out, u_idx_v, u_rows, sem):
    wid = lax.axis_index("s") * NC + lax.axis_index("c")
    u_base = wid * U_PER_W
    pltpu.sync_copy(u_idx_hbm.at[pl.ds(u_base, U_PER_W)], u_idx_v)
    pltpu.async_copy(user_table.at[u_idx_v], u_rows, sem).wait()
    pltpu.sync_copy(u_rows, u_out.at[pl.ds(u_base, U_PER_W)])


def _skill_body(s_idx_hbm, skill_table, s_out3, s_idx_all, rows,
                gsems, wsems):
    wid = lax.axis_index("s") * NC + lax.axis_index("c")
    u_base = wid * U_PER_W
    s_base = wid * S_PER_W

    def s_gather(c, b):
        return pltpu.make_async_copy(
            skill_table.at[s_idx_all.at[pl.ds(c * S_CHUNK, S_CHUNK)]],
            rows[b], gsems[b])

    def s_write_start(c, b):
        user0 = u_base + c * USERS_PER_CHUNK
        for u in range(USERS_PER_CHUNK):
            pltpu.make_async_copy(
                rows[b].at[pl.ds(u * HIST, HIST)],
                s_out3.at[user0 + u], wsems[b]).start()

    def s_write_wait(b):
        for u in range(USERS_PER_CHUNK):
            pltpu.make_async_copy(
                rows[b].at[pl.ds(u * HIST, HIST)],
                s_out3.at[0], wsems[b]).wait()

    pltpu.sync_copy(s_idx_hbm.at[pl.ds(s_base, S_PER_W)], s_idx_all)
    for b in range(NB):
        s_gather(b, b).start()

    def outer(g, carry):
        for b in range(NB):
            c = g * NB + b
            s_gather(c, b).wait()
            s_write_start(c, b)
            s_write_wait(b)
            s_gather(c + NB, b).start()
        return carry

    lax.fori_loop(0, (S_STEPS - NB) // NB, outer, 0)

    for b in range(NB):
        c = S_STEPS - NB + b
        s_gather(c, b).wait()
        s_write_start(c, b)
    for b in range(NB):
        s_write_wait(b)


def _run(user_indices, query_skills_flat, user_table, skill_table):
    s_emb = pl.kernel(
        lambda si, st, so, sidx, r0, r1, r2, r3, g0, g1, g2, g3,
               w0, w1, w2, w3: _skill_body(
                   si, st, so, sidx, [r0, r1, r2, r3],
                   [g0, g1, g2, g3], [w0, w1, w2, w3]),
        out_type=jax.ShapeDtypeStruct((BATCH, HIST, EMBED_DIM), jnp.float32),
        mesh=_mesh(),
        scratch_types=(
            [pltpu.VMEM((S_PER_W,), jnp.int32)]
            + [pltpu.VMEM((S_CHUNK, EMBED_DIM), jnp.float32)
               for _ in range(NB)]
            + [pltpu.SemaphoreType.DMA for _ in range(2 * NB)]
        ),
        compiler_params=pltpu.CompilerParams(use_tc_tiling_on_sc=False),
    )(query_skills_flat, skill_table)

    u_emb = pl.kernel(
        _user_body,
        out_type=jax.ShapeDtypeStruct((BATCH, EMBED_DIM), jnp.float32),
        mesh=_mesh(),
        scratch_types=[
            pltpu.VMEM((U_PER_W,), jnp.int32),
            pltpu.VMEM((U_PER_W, EMBED_DIM), jnp.float32),
            pltpu.SemaphoreType.DMA,
        ],
        compiler_params=pltpu.CompilerParams(use_tc_tiling_on_sc=False),
    )(user_indices, user_table)

    return (u_emb, s_emb)


_RUN_JITTED = None


def _get_run():
    global _RUN_JITTED
    if _RUN_JITTED is None:
        sharding = jax.sharding.SingleDeviceSharding(jax.devices()[0])
        out_formats = (
            Format(Layout(major_to_minor=(0, 1), tiling=()), sharding),
            Format(Layout(major_to_minor=(0, 1, 2), tiling=()), sharding),
        )
        _RUN_JITTED = jax.jit(_run, out_shardings=out_formats)
    return _RUN_JITTED


def kernel(user_indices, query_skills, user_table, skill_table):
    u_idx = user_indices.astype(jnp.int32)
    s_idx = query_skills.astype(jnp.int32).reshape(-1)
    return _get_run()(u_idx, s_idx, user_table, skill_table)
